# Initial kernel scaffold; baseline (speedup 1.0000x reference)
import jax
import jax.numpy as jnp
from jax.experimental import pallas as pl


def kernel(x, edge_index, Wq, Wk, Wm, Wv, Wo):
    raise NotImplementedError("write your pallas kernel here")


# speedup vs baseline: 5.5145x; 5.5145x over previous
import jax
import jax.numpy as jnp
from jax import lax
from jax.experimental import pallas as pl
from jax.experimental.pallas import tpu as pltpu
from jax.experimental.pallas import tpu_sc as plsc

N = 10000
E = 160000
D = 256
HC = 256
INV_SQRT_C = 0.17677669529663687

NC = 2
NS = 16
NW = NC * NS

E_PAD = 163840
EPW = E_PAD // NW
BE = 128
NB_B = EPW // BE

RPN = 320
N_PAD = NW * RPN
SB = 512
SEL = SB + 16

_L = 16


def _proj_body(x_ref, wq_ref, wk_ref, wm_ref, wv_ref, q_ref, k_ref, m_ref, v_ref):
    xb = x_ref[...]
    q_ref[...] = jnp.dot(xb, wq_ref[...], preferred_element_type=jnp.float32)
    k_ref[...] = jnp.dot(xb, wk_ref[...], preferred_element_type=jnp.float32)
    m_ref[...] = jnp.dot(xb, wm_ref[...], preferred_element_type=jnp.float32)
    v_ref[...] = jnp.dot(xb, wv_ref[...], preferred_element_type=jnp.float32)


def _project(x, Wq, Wk, Wm, Wv):
    nblk = 10
    bm = N // nblk
    w_spec = pl.BlockSpec((D, HC), lambda b: (0, 0))
    o_spec = pl.BlockSpec((bm, HC), lambda b: (b, 0))
    return pl.pallas_call(
        _proj_body,
        grid=(nblk,),
        in_specs=[pl.BlockSpec((bm, D), lambda b: (b, 0)), w_spec, w_spec, w_spec, w_spec],
        out_specs=[o_spec, o_spec, o_spec, o_spec],
        out_shape=[jax.ShapeDtypeStruct((N, HC), jnp.float32)] * 4,
    )(x, Wq, Wk, Wm, Wv)


def _scores_body(rowp, colp, q_hbm, k_hbm, exps_out,
                 rowv, colv, qbuf, kbuf, expv, sem):
    c = lax.axis_index("c")
    s = lax.axis_index("s")
    wid = c * NS + s
    lanes = lax.iota(jnp.int32, _L)

    def block(b, _):
        off = wid * EPW + b * BE
        pltpu.sync_copy(rowp.at[pl.ds(off, BE)], rowv)
        pltpu.sync_copy(colp.at[pl.ds(off, BE)], colv)
        cp1 = pltpu.async_copy(q_hbm.at[rowv], qbuf, sem)
        cp2 = pltpu.async_copy(k_hbm.at[colv], kbuf, sem)
        cp1.wait()
        cp2.wait()

        def group(g, _g):
            svec = jnp.zeros((_L,), jnp.float32)
            for l in range(_L):
                e = g * _L + l
                acc = qbuf[e, pl.ds(0, _L)] * kbuf[e, pl.ds(0, _L)]
                for j in range(1, HC // _L):
                    acc += qbuf[e, pl.ds(j * _L, _L)] * kbuf[e, pl.ds(j * _L, _L)]
                sc = jnp.sum(acc)
                svec = jnp.where(lanes == l, sc, svec)
            expv[pl.ds(g * _L, _L)] = jnp.exp(svec * INV_SQRT_C)
            return _g

        lax.fori_loop(0, BE // _L, group, 0)
        pltpu.sync_copy(expv, exps_out.at[pl.ds(off, BE)])
        return _

    lax.fori_loop(0, NB_B, block, 0)


def _scores(rowb, colp, Q, K):
    mesh = plsc.VectorSubcoreMesh(
        core_axis_name="c", subcore_axis_name="s", num_cores=NC, num_subcores=NS)
    fn = pl.kernel(
        _scores_body,
        mesh=mesh,
        out_type=jax.ShapeDtypeStruct((E_PAD,), jnp.float32),
        scratch_types=[
            pltpu.VMEM((BE,), jnp.int32),
            pltpu.VMEM((BE,), jnp.int32),
            pltpu.VMEM((BE, HC), jnp.float32),
            pltpu.VMEM((BE, HC), jnp.float32),
            pltpu.VMEM((BE,), jnp.float32),
            pltpu.SemaphoreType.DMA,
        ],
        compiler_params=pltpu.CompilerParams(needs_layout_passes=False),
    )
    return fn(rowb, colp, Q, K)


def _agg_body(rowp, colp, m_hbm, exps, agg_out,
              rowv, colv, expv, selrel, selcol, selexp, colg, mbuf, agg, den, sem):
    c = lax.axis_index("c")
    s = lax.axis_index("s")
    wid = c * NS + s
    t0 = wid * RPN
    lanes = lax.iota(jnp.int32, _L)
    zf = jnp.zeros((_L,), jnp.float32)

    def zero_row(r, _):
        for j in range(HC // _L):
            agg[r, pl.ds(j * _L, _L)] = zf
        den[r % (RPN // _L + 4), :] = zf
        return _
    lax.fori_loop(0, RPN + 8, zero_row, 0)

    def scan_block(sb, nsel0):
        off = sb * SB
        pltpu.sync_copy(rowp.at[pl.ds(off, SB)], rowv)
        pltpu.sync_copy(colp.at[pl.ds(off, SB)], colv)
        pltpu.sync_copy(exps.at[pl.ds(off, SB)], expv)

        nsel = nsel0 * 0
        for g in range(SB // _L):
            sl = pl.ds(g * _L, _L)
            rel = rowv[sl] - t0
            m = (rel >= 0) & (rel < RPN)
            plsc.store_compressed(selrel.at[pl.ds(nsel, _L)], rel, mask=m)
            plsc.store_compressed(selcol.at[pl.ds(nsel, _L)], colv[sl], mask=m)
            plsc.store_compressed(selexp.at[pl.ds(nsel, _L)], expv[sl], mask=m)
            nsel = nsel + plsc.all_reduce_population_count(m)[0]

        selrel[pl.ds(nsel, _L)] = jnp.full((_L,), RPN, jnp.int32)
        selcol[pl.ds(nsel, _L)] = jnp.zeros((_L,), jnp.int32)
        selexp[pl.ds(nsel, _L)] = zf
        ngrp = (nsel + _L - 1) // _L

        def flush(k, _):
            colg[...] = selcol[pl.ds(k * _L, _L)]
            pltpu.async_copy(m_hbm.at[colg], mbuf, sem).wait()
            rel16 = selrel[pl.ds(k * _L, _L)]
            e16 = selexp[pl.ds(k * _L, _L)]
            for l in range(_L):
                r = rel16[l]
                p = e16[l]
                rg = r // _L
                den[rg, :] = den[rg, :] + jnp.where(lanes == r % _L, p, 0.0)
                for j in range(HC // _L):
                    jl = pl.ds(j * _L, _L)
                    agg[r, jl] = agg[r, jl] + p * mbuf[l, jl]
            return _

        lax.fori_loop(0, ngrp, flush, 0)
        return nsel0 * 0

    lax.fori_loop(0, E_PAD // SB, scan_block, 0)

    def rowg(rg, _):
        inv = 1.0 / (den[rg, :] + 1e-16)
        for l in range(_L):
            rr = rg * _L + l
            p = inv[l]
            for j in range(HC // _L):
                jl = pl.ds(j * _L, _L)
                agg[rr, jl] = agg[rr, jl] * p
        return _
    lax.fori_loop(0, RPN // _L, rowg, 0)
    pltpu.sync_copy(agg.at[pl.ds(0, RPN), pl.ds(0, HC)],
                    agg_out.at[pl.ds(t0, RPN), :])


def _messages(rowc, colp, M, exps):
    mesh = plsc.VectorSubcoreMesh(
        core_axis_name="c", subcore_axis_name="s", num_cores=NC, num_subcores=NS)
    fn = pl.kernel(
        _agg_body,
        mesh=mesh,
        out_type=jax.ShapeDtypeStruct((N_PAD, HC), jnp.float32),
        scratch_types=[
            pltpu.VMEM((SB,), jnp.int32),
            pltpu.VMEM((SB,), jnp.int32),
            pltpu.VMEM((SB,), jnp.float32),
            pltpu.VMEM((SEL,), jnp.int32),
            pltpu.VMEM((SEL,), jnp.int32),
            pltpu.VMEM((SEL,), jnp.float32),
            pltpu.VMEM((_L,), jnp.int32),
            pltpu.VMEM((_L, HC), jnp.float32),
            pltpu.VMEM((RPN + 8, HC), jnp.float32),
            pltpu.VMEM((RPN // _L + 4, _L), jnp.float32),
            pltpu.SemaphoreType.DMA,
        ],
        compiler_params=pltpu.CompilerParams(needs_layout_passes=False),
    )
    return fn(rowc, colp, M, exps)


def _out_body(v_ref, agg_ref, wv_ref, wa_ref, o_ref):
    o_ref[...] = (
        jnp.dot(v_ref[...], wv_ref[...], preferred_element_type=jnp.float32)
        + jnp.dot(agg_ref[...], wa_ref[...], preferred_element_type=jnp.float32)
    )


def _output(V, agg, Wo_V, Wo_A):
    nblk = 10
    bm = N // nblk
    return pl.pallas_call(
        _out_body,
        grid=(nblk,),
        in_specs=[
            pl.BlockSpec((bm, HC), lambda b: (b, 0)),
            pl.BlockSpec((bm, HC), lambda b: (b, 0)),
            pl.BlockSpec((HC, D), lambda b: (0, 0)),
            pl.BlockSpec((HC, D), lambda b: (0, 0)),
        ],
        out_specs=pl.BlockSpec((bm, D), lambda b: (b, 0)),
        out_shape=jax.ShapeDtypeStruct((N, D), jnp.float32),
    )(V, agg, Wo_V, Wo_A)


def kernel(x, edge_index, Wq, Wk, Wm, Wv, Wo):
    row = edge_index[0].astype(jnp.int32)
    col = edge_index[1].astype(jnp.int32)
    npad = E_PAD - E
    rowb = jnp.concatenate([row, jnp.zeros((npad,), jnp.int32)])
    rowc = jnp.concatenate([row, jnp.full((npad,), N, jnp.int32)])
    colp = jnp.concatenate([col, jnp.zeros((npad,), jnp.int32)])

    Wo_r = Wo.reshape(HC // 32, 2, 32, D)
    Wo_V = Wo_r[:, 0].reshape(HC, D)
    Wo_A = Wo_r[:, 1].reshape(HC, D)

    Q, K, M, V = _project(x, Wq, Wk, Wm, Wv)
    exps = _scores(rowb, colp, Q, K)
    agg = _messages(rowc, colp, M, exps)
    return _output(V, agg[:N], Wo_V, Wo_A)


# speedup vs baseline: 8.4721x; 1.5363x over previous
import jax
import jax.numpy as jnp
from jax import lax
from jax.experimental import pallas as pl
from jax.experimental.pallas import tpu as pltpu
from jax.experimental.pallas import tpu_sc as plsc

N = 10000
E = 160000
D = 256
HC = 256
INV_SQRT_C = 0.17677669529663687

NC = 2
NS = 16
NW = NC * NS

E_PAD = 163840
EPW = E_PAD // NW
BE = 128
NB_B = EPW // BE

RPN = 320
N_PAD = NW * RPN
SB = 512
SEL = SB + 16
NB_C = E_PAD // SB

_L = 16


def _proj_body(x_ref, wq_ref, wk_ref, wm_ref, wv_ref, q_ref, k_ref, m_ref, v_ref):
    xb = x_ref[...]
    q_ref[...] = jnp.dot(xb, wq_ref[...], preferred_element_type=jnp.float32)
    k_ref[...] = jnp.dot(xb, wk_ref[...], preferred_element_type=jnp.float32)
    m_ref[...] = jnp.dot(xb, wm_ref[...], preferred_element_type=jnp.float32)
    v_ref[...] = jnp.dot(xb, wv_ref[...], preferred_element_type=jnp.float32)


def _project(x, Wq, Wk, Wm, Wv):
    nblk = 10
    bm = N // nblk
    w_spec = pl.BlockSpec((D, HC), lambda b: (0, 0))
    o_spec = pl.BlockSpec((bm, HC), lambda b: (b, 0))
    return pl.pallas_call(
        _proj_body,
        grid=(nblk,),
        in_specs=[pl.BlockSpec((bm, D), lambda b: (b, 0)), w_spec, w_spec, w_spec, w_spec],
        out_specs=[o_spec, o_spec, o_spec, o_spec],
        out_shape=[jax.ShapeDtypeStruct((N, HC), jnp.float32)] * 4,
    )(x, Wq, Wk, Wm, Wv)


def _scores_body(rowp, colp, q_hbm, k_hbm, exps_out,
                 rowv, colv, qbuf, kbuf, expv, sem):
    c = lax.axis_index("c")
    s = lax.axis_index("s")
    wid = c * NS + s
    lanes = lax.iota(jnp.int32, _L)

    def block(b, _):
        off = wid * EPW + b * BE
        pltpu.sync_copy(rowp.at[pl.ds(off, BE)], rowv)
        pltpu.sync_copy(colp.at[pl.ds(off, BE)], colv)
        pltpu.async_copy(q_hbm.at[rowv], qbuf, sem).wait()
        pltpu.async_copy(k_hbm.at[colv], kbuf, sem).wait()

        def group(g, _g):
            svec = jnp.zeros((_L,), jnp.float32)
            for l in range(_L):
                e = g * _L + l
                acc = qbuf[e, pl.ds(0, _L)] * kbuf[e, pl.ds(0, _L)]
                for j in range(1, HC // _L):
                    acc += qbuf[e, pl.ds(j * _L, _L)] * kbuf[e, pl.ds(j * _L, _L)]
                sc = jnp.sum(acc)
                svec = jnp.where(lanes == l, sc, svec)
            expv[pl.ds(g * _L, _L)] = jnp.exp(svec * INV_SQRT_C)
            return _g

        lax.fori_loop(0, BE // _L, group, 0)
        pltpu.sync_copy(expv, exps_out.at[pl.ds(off, BE)])
        return _

    lax.fori_loop(0, NB_B, block, 0)


def _scores(rowb, colp, Q, K):
    mesh = plsc.VectorSubcoreMesh(
        core_axis_name="c", subcore_axis_name="s", num_cores=NC, num_subcores=NS)
    fn = pl.kernel(
        _scores_body,
        mesh=mesh,
        out_type=jax.ShapeDtypeStruct((E_PAD,), jnp.float32),
        scratch_types=[
            pltpu.VMEM((BE,), jnp.int32),
            pltpu.VMEM((BE,), jnp.int32),
            pltpu.VMEM((BE, HC), jnp.float32),
            pltpu.VMEM((BE, HC), jnp.float32),
            pltpu.VMEM((BE,), jnp.float32),
            pltpu.SemaphoreType.DMA,
        ],
        compiler_params=pltpu.CompilerParams(needs_layout_passes=False),
    )
    return fn(rowb, colp, Q, K)


def _agg_body(ec, exps, m_hbm, agg_out,
              idxv, expv, selrel, selcol, selexp,
              mbuf, agg, den, gsem):
    c = lax.axis_index("c")
    s = lax.axis_index("s")
    wid = c * NS + s
    t0 = wid * RPN
    lanes = lax.iota(jnp.int32, _L)
    zf = jnp.zeros((_L,), jnp.float32)
    zi = jnp.zeros((_L,), jnp.int32)

    def zero_row(r, _):
        for j in range(HC // _L):
            agg[r, pl.ds(j * _L, _L)] = zf
        den[r % (RPN // _L + 4), :] = zf
        return _
    lax.fori_loop(0, RPN + 8, zero_row, 0)
    def zero_sel(k, _):
        selcol[pl.ds(k * _L, _L)] = zi
        return _
    lax.fori_loop(0, SEL // _L, zero_sel, 0)

    def scan(nsel0):
        nsel = nsel0
        for g in range(SB // _L):
            sl = pl.ds(g * _L, _L)
            rel = idxv[0, sl] - t0
            m = (rel >= 0) & (rel < RPN)
            plsc.store_compressed(selrel.at[pl.ds(nsel, _L)], rel, mask=m)
            plsc.store_compressed(selcol.at[pl.ds(nsel, _L)], idxv[1, sl], mask=m)
            plsc.store_compressed(selexp.at[pl.ds(nsel, _L)], expv[sl], mask=m)
            nsel = nsel + plsc.all_reduce_population_count(m)[0]
        return nsel

    def fma_edges(e0, e1, mshift):
        def edge(e, _):
            ev = jnp.full((_L,), e, jnp.int32)
            rv = plsc.load_gather(selrel, [ev])
            pv = plsc.load_gather(selexp, [ev])
            r = rv[0]
            rg = r // _L
            den[rg, :] = den[rg, :] + jnp.where(lanes == r % _L, pv, 0.0)
            mr = e - mshift
            for j in range(HC // _L):
                jl = pl.ds(j * _L, _L)
                agg[r, jl] = agg[r, jl] + pv * mbuf[mr, jl]
            return _
        lax.fori_loop(e0, e1, edge, 0)

    def block(sb, _):
        off = sb * SB
        pltpu.sync_copy(ec.at[0, pl.ds(off, SB)], idxv.at[0])
        pltpu.sync_copy(ec.at[1, pl.ds(off, SB)], idxv.at[1])
        pltpu.sync_copy(exps.at[pl.ds(off, SB)], expv)
        nsel = scan(0)
        ngrp = (nsel + _L - 1) // _L

        def grp(k, _g):
            pltpu.async_copy(
                m_hbm.at[selcol.at[pl.ds(k * _L, _L)]], mbuf, gsem).wait()
            fma_edges(k * _L, jnp.minimum(nsel, (k + 1) * _L), k * _L)
            return _g
        lax.fori_loop(0, ngrp, grp, 0)
        return _

    lax.fori_loop(0, NB_C, block, 0)

    def rowg(r, _):
        dv = plsc.load_gather(
            den, [jnp.full((_L,), r // _L, jnp.int32),
                  jnp.full((_L,), r % _L, jnp.int32)])
        inv = 1.0 / (dv + 1e-16)
        for j in range(HC // _L):
            jl = pl.ds(j * _L, _L)
            agg[r, jl] = agg[r, jl] * inv
        return _
    lax.fori_loop(0, RPN, rowg, 0)
    pltpu.sync_copy(agg.at[pl.ds(0, RPN), pl.ds(0, HC)],
                    agg_out.at[pl.ds(t0, RPN), :])


def _messages(ec, M, exps):
    mesh = plsc.VectorSubcoreMesh(
        core_axis_name="c", subcore_axis_name="s", num_cores=NC, num_subcores=NS)
    fn = pl.kernel(
        _agg_body,
        mesh=mesh,
        out_type=jax.ShapeDtypeStruct((N_PAD, HC), jnp.float32),
        scratch_types=[
            pltpu.VMEM((2, SB), jnp.int32),
            pltpu.VMEM((SB,), jnp.float32),
            pltpu.VMEM((SEL,), jnp.int32),
            pltpu.VMEM((SEL,), jnp.int32),
            pltpu.VMEM((SEL,), jnp.float32),
            pltpu.VMEM((_L, HC), jnp.float32),
            pltpu.VMEM((RPN + 8, HC), jnp.float32),
            pltpu.VMEM((RPN // _L + 4, _L), jnp.float32),
            pltpu.SemaphoreType.DMA,
        ],
        compiler_params=pltpu.CompilerParams(needs_layout_passes=False),
    )
    return fn(ec, exps, M)


def _out_body(v_ref, agg_ref, wv_ref, wa_ref, o_ref):
    o_ref[...] = (
        jnp.dot(v_ref[...], wv_ref[...], preferred_element_type=jnp.float32)
        + jnp.dot(agg_ref[...], wa_ref[...], preferred_element_type=jnp.float32)
    )


def _output(V, agg, Wo_V, Wo_A):
    nblk = 10
    bm = N // nblk
    return pl.pallas_call(
        _out_body,
        grid=(nblk,),
        in_specs=[
            pl.BlockSpec((bm, HC), lambda b: (b, 0)),
            pl.BlockSpec((bm, HC), lambda b: (b, 0)),
            pl.BlockSpec((HC, D), lambda b: (0, 0)),
            pl.BlockSpec((HC, D), lambda b: (0, 0)),
        ],
        out_specs=pl.BlockSpec((bm, D), lambda b: (b, 0)),
        out_shape=jax.ShapeDtypeStruct((N, D), jnp.float32),
    )(V, agg, Wo_V, Wo_A)


def kernel(x, edge_index, Wq, Wk, Wm, Wv, Wo):
    row = edge_index[0].astype(jnp.int32)
    col = edge_index[1].astype(jnp.int32)
    npad = E_PAD - E
    rowb = jnp.concatenate([row, jnp.zeros((npad,), jnp.int32)])
    rowc = jnp.concatenate([row, jnp.full((npad,), N, jnp.int32)])
    colp = jnp.concatenate([col, jnp.zeros((npad,), jnp.int32)])

    Wo_r = Wo.reshape(HC // 32, 2, 32, D)
    Wo_V = Wo_r[:, 0].reshape(HC, D)
    Wo_A = Wo_r[:, 1].reshape(HC, D)

    Q, K, M, V = _project(x, Wq, Wk, Wm, Wv)
    exps = _scores(rowb, colp, Q, K)
    ec = jnp.stack([rowc, colp])
    agg = _messages(ec, M, exps)
    return _output(V, agg[:N], Wo_V, Wo_A)
